# baseline (speedup 1.0000x reference)
import functools

import jax
import jax.numpy as jnp
from jax import lax
from jax.experimental import pallas as pl
from jax.experimental.pallas import tpu as pltpu
from jax.experimental.pallas import tpu_sc as plsc

N_COMP = 64
ROWS = 16384
NC, NS, LANES = 2, 16, 16
NW = NC * NS
ROWS_W = ROWS // NW
WORDS_W = ROWS_W * N_COMP
GROUPS = ROWS_W // LANES


@functools.partial(
    pl.kernel,
    out_type=(
        jax.ShapeDtypeStruct((NW, N_COMP), jnp.float32),
        jax.ShapeDtypeStruct((NW, LANES), jnp.float32),
    ),
    mesh=plsc.VectorSubcoreMesh(
        core_axis_name="c", subcore_axis_name="s",
        num_cores=NC, num_subcores=NS,
    ),
    scratch_types=(
        pltpu.VMEM((WORDS_W,), jnp.float32),
        pltpu.VMEM((N_COMP,), jnp.float32),
        pltpu.VMEM((LANES,), jnp.float32),
        pltpu.SemaphoreType.DMA,
        pltpu.SemaphoreType.DMA,
    ),
    compiler_params=pltpu.CompilerParams(needs_layout_passes=False),
)
def _sc_pass(zc_hbm, pop_hbm, conf_hbm, buf, pop, conf, sem0, sem1):
    wid = lax.axis_index("s") * NC + lax.axis_index("c")
    half = WORDS_W // 2
    cp0 = pltpu.async_copy(
        zc_hbm.at[pl.ds(wid * WORDS_W, half)], buf.at[pl.ds(0, half)], sem0)
    cp1 = pltpu.async_copy(
        zc_hbm.at[pl.ds(wid * WORDS_W + half, half)],
        buf.at[pl.ds(half, half)], sem1)

    zeros16 = jnp.zeros((LANES,), jnp.float32)
    for k in range(N_COMP // LANES):
        pop[pl.ds(k * LANES, LANES)] = zeros16

    row_off = lax.iota(jnp.int32, LANES) * N_COMP
    ones16 = jnp.ones((LANES,), jnp.float32)
    zeros16i = jnp.zeros((LANES,), jnp.int32)

    def g_body(g, conf_acc):
        base = row_off + g * (LANES * N_COMP)
        ms, aas = [], []
        for k in range(4):
            m = jnp.full((LANES,), -1.0, jnp.float32)
            a = zeros16i
            for i in range(LANES):
                c = k * LANES + i
                v = plsc.load_gather(buf, [base + c])
                upd = v > m
                m = jnp.where(upd, v, m)
                a = jnp.where(upd, c, a)
            ms.append(m)
            aas.append(a)
        g1 = ms[1] > ms[0]
        m01 = jnp.where(g1, ms[1], ms[0])
        a01 = jnp.where(g1, aas[1], aas[0])
        g2 = ms[3] > ms[2]
        m23 = jnp.where(g2, ms[3], ms[2])
        a23 = jnp.where(g2, aas[3], aas[2])
        g3 = m23 > m01
        m = jnp.where(g3, m23, m01)
        a = jnp.where(g3, a23, a01)
        plsc.store_scatter(pop, [a], ones16)
        return conf_acc + m

    cp0.wait()
    conf_acc = lax.fori_loop(0, GROUPS // 2, g_body, zeros16)
    cp1.wait()
    conf_acc = lax.fori_loop(GROUPS // 2, GROUPS, g_body, conf_acc)
    conf[...] = conf_acc
    pltpu.sync_copy(pop, pop_hbm.at[wid])
    pltpu.sync_copy(conf, conf_hbm.at[wid])


def _merge_body(pop_ref, conf_ref, np_ref, cm_ref):
    present = jnp.max(pop_ref[...], axis=0, keepdims=True)
    num_pop = jnp.sum(jnp.where(present > 0.0, 1.0, 0.0))
    np_ref[...] = num_pop.reshape(1, 1)
    cm_ref[...] = (jnp.sum(conf_ref[...]) * (1.0 / ROWS)).reshape(1, 1)


_merge = pl.pallas_call(
    _merge_body,
    out_shape=(
        jax.ShapeDtypeStruct((1, 1), jnp.float32),
        jax.ShapeDtypeStruct((1, 1), jnp.float32),
    ),
)


def kernel(z, z_cat):
    zc = z_cat.reshape(ROWS * N_COMP)
    pop_part, conf_part = _sc_pass(zc)
    num_pop, conf_mean = _merge(pop_part, conf_part)
    return (z, num_pop[0, 0], conf_mean[0, 0])


# speedup vs baseline: 1.1246x; 1.1246x over previous
import functools

import jax
import jax.numpy as jnp
from jax import lax
from jax.experimental import pallas as pl
from jax.experimental.pallas import tpu as pltpu
from jax.experimental.pallas import tpu_sc as plsc

N_COMP = 64
ROWS = 16384
NC, NS, LANES = 2, 16, 16
NW = NC * NS
ROWS_W = ROWS // NW
WORDS_W = ROWS_W * N_COMP
GROUPS = ROWS_W // LANES


@functools.partial(
    pl.kernel,
    out_type=(
        jax.ShapeDtypeStruct((NW, N_COMP), jnp.float32),
        jax.ShapeDtypeStruct((NW, LANES), jnp.float32),
    ),
    mesh=plsc.VectorSubcoreMesh(
        core_axis_name="c", subcore_axis_name="s",
        num_cores=NC, num_subcores=NS,
    ),
    scratch_types=(
        pltpu.VMEM((ROWS_W, N_COMP), jnp.float32),
        pltpu.VMEM((N_COMP,), jnp.float32),
        pltpu.VMEM((LANES,), jnp.float32),
        pltpu.SemaphoreType.DMA,
        pltpu.SemaphoreType.DMA,
    ),
    compiler_params=pltpu.CompilerParams(needs_layout_passes=False),
)
def _sc_pass(zc_hbm, pop_hbm, conf_hbm, buf, pop, conf, sem0, sem1):
    wid = lax.axis_index("s") * NC + lax.axis_index("c")
    r0 = wid * ROWS_W
    hrows = ROWS_W // 2
    cp0 = pltpu.async_copy(
        zc_hbm.at[pl.ds(r0, hrows), :], buf.at[pl.ds(0, hrows), :], sem0)
    cp1 = pltpu.async_copy(
        zc_hbm.at[pl.ds(r0 + hrows, hrows), :],
        buf.at[pl.ds(hrows, hrows), :], sem1)

    zeros16 = jnp.zeros((LANES,), jnp.float32)
    for k in range(N_COMP // LANES):
        pop[pl.ds(k * LANES, LANES)] = zeros16

    lanes16 = lax.iota(jnp.int32, LANES)
    ones16 = jnp.ones((LANES,), jnp.float32)
    mask0 = lanes16 == 0

    def row_calc(r, conf_acc):
        v0 = buf[r, pl.ds(0, LANES)]
        v1 = buf[r, pl.ds(LANES, LANES)]
        v2 = buf[r, pl.ds(2 * LANES, LANES)]
        v3 = buf[r, pl.ds(3 * LANES, LANES)]
        g1 = v1 > v0
        m01 = jnp.where(g1, v1, v0)
        k01 = jnp.where(g1, 1, 0)
        g2 = v3 > v2
        m23 = jnp.where(g2, v3, v2)
        k23 = jnp.where(g2, 3, 2)
        g3 = m23 > m01
        mm = jnp.where(g3, m23, m01)
        kk = jnp.where(g3, k23, k01)
        m_row = jnp.max(mm)
        m_vec = jnp.full((LANES,), m_row, jnp.float32)
        cand = kk * LANES + lanes16
        a_row = jnp.min(jnp.where(mm == m_vec, cand, N_COMP))
        plsc.store_scatter(
            pop, [jnp.full((LANES,), a_row, jnp.int32)], ones16, mask=mask0)
        return conf_acc + m_vec

    def g_body(g, conf_acc):
        r_base = g * LANES
        for j in range(LANES):
            conf_acc = row_calc(r_base + j, conf_acc)
        return conf_acc

    cp0.wait()
    conf_acc = lax.fori_loop(0, GROUPS // 2, g_body, zeros16)
    cp1.wait()
    conf_acc = lax.fori_loop(GROUPS // 2, GROUPS, g_body, conf_acc)
    conf[...] = conf_acc
    pltpu.sync_copy(pop, pop_hbm.at[wid])
    pltpu.sync_copy(conf, conf_hbm.at[wid])


def _merge_body(pop_ref, conf_ref, np_ref, cm_ref):
    present = jnp.max(pop_ref[...], axis=0, keepdims=True)
    num_pop = jnp.sum(jnp.where(present > 0.0, 1.0, 0.0))
    np_ref[...] = num_pop.reshape(1, 1)
    cm_ref[...] = (jnp.sum(conf_ref[...]) * (1.0 / (ROWS * LANES))).reshape(1, 1)


_merge = pl.pallas_call(
    _merge_body,
    out_shape=(
        jax.ShapeDtypeStruct((1, 1), jnp.float32),
        jax.ShapeDtypeStruct((1, 1), jnp.float32),
    ),
)


def kernel(z, z_cat):
    pop_part, conf_part = _sc_pass(z_cat)
    num_pop, conf_mean = _merge(pop_part, conf_part)
    return (z, num_pop[0, 0], conf_mean[0, 0])
